# baseline (speedup 1.0000x reference)
import jax
import jax.numpy as jnp
from jax import lax
from jax.experimental import pallas as pl
from jax.experimental.pallas import tpu as pltpu
from jax.experimental.pallas import tpu_sc as plsc

N_STY = 256
N_ADR = 371
NKEY = N_STY * N_ADR
NIN = 16384
NR = NIN * 3
D = 128
K = 12415
TROWS = K + 1
NPASS = 4
CHUNK = NR // 16
BLK = 64
NBLK = CHUNK // BLK
TRASH_OUT = NR
TSPAN = TROWS // 16
CNTROWS = 784
CSPAN = CNTROWS // 16
_SPANS = [(i * 64, 64) for i in range(12)] + [(768, 8)]


def _body(sty_hbm, adr_hbm, feat_hbm, out_hbm,
          table_sh, counts_sh,
          keys_v, feat_v, idx_v, oidx_v, cbuf_v, sty_v, adr_v):
  c = lax.axis_index("c")
  s = lax.axis_index("s")
  iota = lax.iota(jnp.int32, 16)
  zf = jnp.zeros((16,), jnp.float32)

  def zero_feat():
    @pl.loop(0, BLK)
    def _(i):
      @pl.loop(0, 8)
      def _(r):
        feat_v[i, pl.ds(r * 16, 16)] = zf

  pltpu.sync_copy(sty_hbm.at[pl.ds(s * (CHUNK // 3), CHUNK // 3)], sty_v)
  pltpu.sync_copy(adr_hbm.at[pl.ds(s * CHUNK, CHUNK)], adr_v)

  @pl.loop(0, CHUNK // 16)
  def _(j):
    base = j * 16
    sidx = (base + iota) // 3
    sty16 = plsc.load_gather(sty_v, [sidx])
    a16 = adr_v[pl.ds(base, 16)]
    keys_v[pl.ds(base, 16)] = sty16 * N_ADR + a16

  zero_feat()
  pltpu.sync_copy(feat_v.at[pl.ds(0, CSPAN)],
                  counts_sh.at[pl.ds(s * CSPAN, CSPAN)])
  plsc.subcore_barrier()

  @pl.loop(0, NBLK)
  def _(b):
    @pl.loop(0, BLK // 16)
    def _(j):
      idx_v[pl.ds(j * 16, 16)] = keys_v[pl.ds(b * BLK + j * 16, 16)] >> 7

    @pl.loop(0, BLK)
    def _(i):
      k16 = plsc.load_gather(keys_v, [jnp.full((16,), b * BLK + i, jnp.int32)])
      kmod = k16 & 127
      for r in range(8):
        feat_v[i, pl.ds(r * 16, 16)] = jnp.where(
            (iota + r * 16) == kmod, 1.0, 0.0).astype(jnp.float32)

    pltpu.sync_copy(feat_v, counts_sh.at[idx_v], add=True)

  plsc.subcore_barrier()

  for p in range(NPASS):
    base_key = (2 * p + c) * K

    zero_feat()
    for off, n in _SPANS:
      pltpu.sync_copy(feat_v.at[pl.ds(0, n)],
                      table_sh.at[pl.ds(s * TSPAN + off, n)])
    plsc.subcore_barrier()

    @pl.loop(0, NBLK)
    def _(b):
      g0 = s * CHUNK + b * BLK
      pltpu.sync_copy(feat_hbm.at[pl.ds(g0, BLK)], feat_v)

      @pl.loop(0, BLK // 16)
      def _(j):
        k16 = keys_v[pl.ds(b * BLK + j * 16, 16)]
        lk = k16 - base_key
        valid = (lk >= 0) & (lk < K)
        idx_v[pl.ds(j * 16, 16)] = jnp.where(valid, lk, K)

      pltpu.sync_copy(feat_v, table_sh.at[idx_v], add=True)

    plsc.subcore_barrier()

    gk0 = base_key + s * TSPAN
    grow0 = gk0 >> 7
    rel0 = gk0 - grow0 * 128
    pltpu.sync_copy(counts_sh.at[pl.ds(grow0, 8)], cbuf_v)
    for off, n in _SPANS:
      row0 = s * TSPAN + off
      pltpu.sync_copy(table_sh.at[pl.ds(row0, n)], feat_v.at[pl.ds(0, n)])

      @pl.loop(0, n)
      def _(i):
        rel = rel0 + off + i
        cnt = plsc.load_gather(
            cbuf_v, [jnp.full((16,), rel >> 7, jnp.int32),
                     jnp.full((16,), rel & 127, jnp.int32)])
        inv = 1.0 / jnp.maximum(cnt, 1.0)

        @pl.loop(0, 8)
        def _(r):
          feat_v[i, pl.ds(r * 16, 16)] = feat_v[i, pl.ds(r * 16, 16)] * inv

      pltpu.sync_copy(feat_v.at[pl.ds(0, n)], table_sh.at[pl.ds(row0, n)])

    plsc.subcore_barrier()

    @pl.loop(0, NBLK)
    def _(b):
      g0 = s * CHUNK + b * BLK

      @pl.loop(0, BLK // 16)
      def _(j):
        k16 = keys_v[pl.ds(b * BLK + j * 16, 16)]
        lk = k16 - base_key
        valid = (lk >= 0) & (lk < K)
        idx_v[pl.ds(j * 16, 16)] = jnp.where(valid, lk, K)
        g = g0 + j * 16 + iota
        oidx_v[pl.ds(j * 16, 16)] = jnp.where(valid, g, TRASH_OUT)

      pltpu.sync_copy(table_sh.at[idx_v], feat_v)
      pltpu.sync_copy(feat_v, out_hbm.at[oidx_v])

    plsc.subcore_barrier()


@jax.jit
def _dynmem(styles, addrs, feats):
  mesh = plsc.VectorSubcoreMesh(
      core_axis_name="c", subcore_axis_name="s", num_cores=2, num_subcores=16)
  f32, i32 = jnp.float32, jnp.int32
  call = pl.kernel(
      _body,
      out_type=jax.ShapeDtypeStruct((NR + 1, D), f32),
      mesh=mesh,
      compiler_params=pltpu.CompilerParams(needs_layout_passes=False),
      scratch_types=[
          pltpu.VMEM_SHARED((TROWS, D), f32),
          pltpu.VMEM_SHARED((CNTROWS, D), f32),
          pltpu.VMEM((CHUNK,), i32),
          pltpu.VMEM((BLK, D), f32),
          pltpu.VMEM((BLK,), i32),
          pltpu.VMEM((BLK,), i32),
          pltpu.VMEM((8, D), f32),
          pltpu.VMEM((CHUNK // 3,), i32),
          pltpu.VMEM((CHUNK,), i32),
      ],
  )
  return call(styles, addrs, feats)


def kernel(style_ids, comp_addrs, comp_feats):
  styles = style_ids.astype(jnp.int32)
  addrs = comp_addrs.reshape(-1).astype(jnp.int32)
  feats = comp_feats.reshape(-1, D)
  out = _dynmem(styles, addrs, feats)
  return out[:NR].reshape(NIN, 3, D)


# speedup vs baseline: 11.8374x; 11.8374x over previous
import jax
import jax.numpy as jnp
from jax import lax
from jax.experimental import pallas as pl
from jax.experimental.pallas import tpu as pltpu
from jax.experimental.pallas import tpu_sc as plsc

N_STY = 256
N_ADR = 371
NKEY = N_STY * N_ADR
NIN = 16384
NR = NIN * 3
D = 128
K = 9600
TROWS = K + 1
NPASS = 5
CHUNK = NR // 16
NBLK = CHUNK // 128
TRASH_OUT = NR
HROWS = 80
ARENA = CHUNK + 5 * 128
PAD_ID = CHUNK


def _body(sty_hbm, adr_hbm, feat_hbm, out_hbm,
          table_sh, slab_sh,
          keys_v, feat_v, idx_v, gidx_v, hist_v, arena_v,
          secoff_v, rowidx_v, sty_v, adr_v):
  c = lax.axis_index("c")
  s = lax.axis_index("s")
  iota = lax.iota(jnp.int32, 16)
  zf = jnp.zeros((16,), jnp.float32)

  pltpu.sync_copy(sty_hbm.at[pl.ds(s * (CHUNK // 3), CHUNK // 3)], sty_v)
  pltpu.sync_copy(adr_hbm.at[pl.ds(s * CHUNK, CHUNK)], adr_v)

  @pl.loop(0, CHUNK // 16)
  def _(j):
    base = j * 16
    sidx = (base + iota) // 3
    sty16 = plsc.load_gather(sty_v, [sidx])
    a16 = adr_v[pl.ds(base, 16)]
    keys_v[pl.ds(base, 16)] = sty16 * N_ADR + a16

  keys_v[pl.ds(CHUNK, 16)] = jnp.full((16,), -1, jnp.int32)

  @pl.loop(0, ARENA // 16)
  def _(i):
    arena_v[pl.ds(i * 16, 16)] = jnp.full((16,), PAD_ID, jnp.int32)

  @pl.loop(0, HROWS // 16)
  def _(m):
    rowidx_v[pl.ds(m * 16, 16)] = iota + m * 16

  start = jnp.int32(0)
  for pp in range(NPASS):
    rtarget = 2 * pp + c

    def scan_body(j, pos, rtarget=rtarget):
      k16 = keys_v[pl.ds(j * 16, 16)]
      m = (k16 // K) == rtarget
      plsc.store_compressed(arena_v.at[pl.ds(pos, 16)], j * 16 + iota, mask=m)
      return pos + jnp.max(plsc.all_reduce_population_count(m))

    end_real = lax.fori_loop(0, CHUNK // 16, scan_body, start)
    secoff_v[2 * pp] = start
    secoff_v[2 * pp + 1] = end_real
    start = ((end_real + 127) >> 7) << 7

  for p in range(NPASS):
    base_key = (2 * p + c) * K
    sec0 = secoff_v[2 * p]
    sec1 = secoff_v[2 * p + 1]
    nb = (sec1 - sec0 + 127) >> 7

    def build_idx(b, base_key=base_key, sec0=sec0):
      @pl.loop(0, 8)
      def _(j):
        id16 = arena_v[pl.ds(sec0 + b * 128 + j * 16, 16)]
        k16 = plsc.load_gather(keys_v, [id16])
        lk = k16 - base_key
        valid = (lk >= 0) & (lk < K)
        idx_v[pl.ds(j * 16, 16)] = jnp.where(valid, lk, K)
        gidx_v[pl.ds(j * 16, 16)] = jnp.where(
            valid, s * CHUNK + id16, TRASH_OUT)

    @pl.loop(0, 128)
    def _(i):
      @pl.loop(0, 8)
      def _(r):
        feat_v[i, pl.ds(r * 16, 16)] = zf

    pltpu.sync_copy(feat_v.at[pl.ds(0, HROWS // 16)],
                    slab_sh.at[pl.ds(s * (HROWS // 16), HROWS // 16)])

    @pl.loop(0, HROWS)
    def _(i):
      @pl.loop(0, 8)
      def _(r):
        hist_v[i, pl.ds(r * 16, 16)] = zf

    @pl.loop(0, NBLK)
    def _(b):
      @pl.when(b < nb)
      def _():
        build_idx(b)
        pltpu.sync_copy(feat_v, table_sh.at[idx_v])

    plsc.subcore_barrier()

    @pl.loop(0, NBLK)
    def _(b):
      @pl.when(b < nb)
      def _():
        build_idx(b)
        pltpu.sync_copy(feat_hbm.at[gidx_v], feat_v)
        pltpu.sync_copy(feat_v, table_sh.at[idx_v], add=True)

    def hist_body(i, carry, base_key=base_key):
      rid16 = plsc.load_gather(arena_v, [jnp.full((16,), i, jnp.int32)])
      k16 = plsc.load_gather(keys_v, [rid16])
      lk = jnp.max(k16) - base_key
      hi = lk >> 7
      off = lk & 112
      lane = lk & 15
      oh = jnp.where(iota == lane, 1.0, 0.0).astype(jnp.float32)
      hist_v[hi, pl.ds(off, 16)] = hist_v[hi, pl.ds(off, 16)] + oh
      return carry

    lax.fori_loop(sec0, sec1, hist_body, jnp.int32(0))
    pltpu.sync_copy(hist_v, slab_sh.at[rowidx_v], add=True)
    plsc.subcore_barrier()

    pltpu.sync_copy(slab_sh, hist_v)

    @pl.loop(0, NBLK)
    def _(b):
      @pl.when(b < nb)
      def _():
        build_idx(b)
        pltpu.sync_copy(table_sh.at[idx_v], feat_v)

        @pl.loop(0, 128)
        def _(i):
          lk16 = plsc.load_gather(idx_v, [jnp.full((16,), i, jnp.int32)])
          cnt = plsc.load_gather(hist_v, [lk16 >> 7, lk16 & 127])
          inv = 1.0 / jnp.maximum(cnt, 1.0)

          @pl.loop(0, 8)
          def _(r):
            feat_v[i, pl.ds(r * 16, 16)] = feat_v[i, pl.ds(r * 16, 16)] * inv

        pltpu.sync_copy(feat_v, out_hbm.at[gidx_v])

    plsc.subcore_barrier()


@jax.jit
def _dynmem(styles, addrs, feats):
  mesh = plsc.VectorSubcoreMesh(
      core_axis_name="c", subcore_axis_name="s", num_cores=2, num_subcores=16)
  f32, i32 = jnp.float32, jnp.int32
  call = pl.kernel(
      _body,
      out_type=jax.ShapeDtypeStruct((NR + 1, D), f32),
      mesh=mesh,
      compiler_params=pltpu.CompilerParams(needs_layout_passes=False),
      scratch_types=[
          pltpu.VMEM_SHARED((TROWS, D), f32),
          pltpu.VMEM_SHARED((HROWS, D), f32),
          pltpu.VMEM((CHUNK + 16,), i32),
          pltpu.VMEM((128, D), f32),
          pltpu.VMEM((128,), i32),
          pltpu.VMEM((128,), i32),
          pltpu.VMEM((HROWS, D), f32),
          pltpu.VMEM((ARENA,), i32),
          pltpu.SMEM((16,), i32),
          pltpu.VMEM((HROWS,), i32),
          pltpu.VMEM((CHUNK // 3,), i32),
          pltpu.VMEM((CHUNK,), i32),
      ],
  )
  return call(styles, addrs, feats)


def kernel(style_ids, comp_addrs, comp_feats):
  styles = style_ids.astype(jnp.int32)
  addrs = comp_addrs.reshape(-1).astype(jnp.int32)
  feats = jnp.concatenate(
      [comp_feats.reshape(-1, D), jnp.zeros((1, D), jnp.float32)], axis=0)
  out = _dynmem(styles, addrs, feats)
  return out[:NR].reshape(NIN, 3, D)


# speedup vs baseline: 18.0429x; 1.5242x over previous
import jax
import jax.numpy as jnp
from jax import lax
from jax.experimental import pallas as pl
from jax.experimental.pallas import tpu as pltpu
from jax.experimental.pallas import tpu_sc as plsc

N_STY = 256
N_ADR = 371
NKEY = N_STY * N_ADR
NIN = 16384
NR = NIN * 3
D = 128
K = 9600
TROWS = K + 1
NPASS = 5
CHUNK = NR // 16
BLK = 96
NBLK = 32
TRASH_OUT = NR
HROWS = 80
ARENA = CHUNK + 5 * BLK
PAD_ID = CHUNK


def _body(sty_hbm, adr_hbm, feat_hbm, out_hbm,
          table_sh, slab_sh,
          keys_v, feat_a, feat_b, hist_v, arena_v, idxb_v, gidxb_v,
          rowidx_v, secoff_s,
          zsem, asem_a, asem_b, osem_a, osem_b):
  c = lax.axis_index("c")
  s = lax.axis_index("s")
  iota = lax.iota(jnp.int32, 16)
  zf = jnp.zeros((16,), jnp.float32)

  pltpu.sync_copy(sty_hbm.at[pl.ds(s * (CHUNK // 3), CHUNK // 3)],
                  keys_v.at[pl.ds(CHUNK - 1008, CHUNK // 3)])
  pltpu.sync_copy(adr_hbm.at[pl.ds(s * CHUNK, CHUNK)],
                  arena_v.at[pl.ds(0, CHUNK)])

  @pl.loop(0, CHUNK // 16)
  def _(j):
    base = j * 16
    sidx = (CHUNK - 1008) + (base + iota) // 3
    sty16 = plsc.load_gather(keys_v, [sidx])
    a16 = arena_v[pl.ds(base, 16)]
    keys_v[pl.ds(base, 16)] = sty16 * N_ADR + a16

  keys_v[pl.ds(CHUNK, 16)] = jnp.full((16,), -1, jnp.int32)

  @pl.loop(0, ARENA // 16)
  def _(i):
    arena_v[pl.ds(i * 16, 16)] = jnp.full((16,), PAD_ID, jnp.int32)

  @pl.loop(0, HROWS // 16)
  def _(m):
    rowidx_v[pl.ds(m * 16, 16)] = iota + m * 16

  start = jnp.int32(0)
  for pp in range(NPASS):
    rtarget = 2 * pp + c

    def scan_body(j, pos, rtarget=rtarget):
      k16 = keys_v[pl.ds(j * 16, 16)]
      m = (k16 // K) == rtarget
      plsc.store_compressed(arena_v.at[pl.ds(pos, 16)], j * 16 + iota, mask=m)
      return pos + jnp.max(plsc.all_reduce_population_count(m))

    end_real = lax.fori_loop(0, CHUNK // 16, scan_body, start)
    secoff_s[2 * pp] = start
    secoff_s[2 * pp + 1] = end_real
    start = ((end_real + BLK - 1) // BLK) * BLK

  for p in range(NPASS):
    base_key = (2 * p + c) * K
    sec0 = secoff_s[2 * p]
    sec1 = secoff_s[2 * p + 1]
    nb = (sec1 - sec0 + BLK - 1) // BLK

    @pl.loop(0, NBLK)
    def _(b, base_key=base_key, sec0=sec0, nb=nb):
      @pl.when(b < nb)
      def _():
        @pl.loop(0, BLK // 16)
        def _(j):
          id16 = arena_v[pl.ds(sec0 + b * BLK + j * 16, 16)]
          k16 = plsc.load_gather(keys_v, [id16])
          lk = k16 - base_key
          valid = (lk >= 0) & (lk < K)
          idxb_v[b, 0, pl.ds(j * 16, 16)] = jnp.where(valid, lk, K)
          gidxb_v[b, 0, pl.ds(j * 16, 16)] = jnp.where(
              valid, s * CHUNK + id16, TRASH_OUT)

    @pl.loop(0, BLK)
    def _(i):
      @pl.loop(0, 8)
      def _(r):
        feat_a[i, pl.ds(r * 16, 16)] = zf

    @pl.loop(0, NBLK)
    def _(b, nb=nb):
      @pl.when(b < nb)
      def _():
        pltpu.async_copy(feat_a, table_sh.at[idxb_v.at[b, 0]], zsem)

    pltpu.sync_copy(feat_a.at[pl.ds(0, HROWS // 16)],
                    slab_sh.at[pl.ds(s * (HROWS // 16), HROWS // 16)])

    @pl.loop(0, HROWS)
    def _(i):
      @pl.loop(0, 8)
      def _(r):
        hist_v[i, pl.ds(r * 16, 16)] = zf

    @pl.loop(0, NBLK)
    def _(b, nb=nb):
      @pl.when(b < nb)
      def _():
        pltpu.make_async_copy(feat_a, table_sh.at[idxb_v.at[0, 0]], zsem).wait()

    plsc.subcore_barrier()

    @pl.loop(0, NBLK // 2)
    def _(t, nb=nb):
      b0 = 2 * t
      b1 = 2 * t + 1

      @pl.when((t > 0) & (b0 < nb))
      def _():
        pltpu.make_async_copy(feat_a, table_sh.at[idxb_v.at[0, 0]], asem_a).wait()

      @pl.when(b0 < nb)
      def _():
        pltpu.sync_copy(feat_hbm.at[gidxb_v.at[b0, 0]], feat_a)
        pltpu.async_copy(feat_a, table_sh.at[idxb_v.at[b0, 0]], asem_a, add=True)

      @pl.when((t > 0) & (b1 < nb))
      def _():
        pltpu.make_async_copy(feat_b, table_sh.at[idxb_v.at[0, 0]], asem_b).wait()

      @pl.when(b1 < nb)
      def _():
        pltpu.sync_copy(feat_hbm.at[gidxb_v.at[b1, 0]], feat_b)
        pltpu.async_copy(feat_b, table_sh.at[idxb_v.at[b1, 0]], asem_b, add=True)

    def hist_body(i, carry, base_key=base_key):
      rid16 = plsc.load_gather(arena_v, [jnp.full((16,), i, jnp.int32)])
      k16 = plsc.load_gather(keys_v, [rid16])
      lk = jnp.max(k16) - base_key
      hi = lk >> 7
      off = lk & 112
      lane = lk & 15
      oh = jnp.where(iota == lane, 1.0, 0.0).astype(jnp.float32)
      hist_v[hi, pl.ds(off, 16)] = hist_v[hi, pl.ds(off, 16)] + oh
      return carry

    lax.fori_loop(sec0, sec1, hist_body, jnp.int32(0))

    @pl.when(nb >= 1)
    def _():
      pltpu.make_async_copy(feat_a, table_sh.at[idxb_v.at[0, 0]], asem_a).wait()

    @pl.when(nb >= 2)
    def _():
      pltpu.make_async_copy(feat_b, table_sh.at[idxb_v.at[0, 0]], asem_b).wait()

    pltpu.sync_copy(hist_v, slab_sh.at[rowidx_v], add=True)
    plsc.subcore_barrier()

    pltpu.sync_copy(slab_sh, hist_v)

    def divide(buf, b):
      @pl.loop(0, BLK)
      def _(i):
        lk16 = plsc.load_gather(
            idxb_v, [jnp.full((16,), b, jnp.int32),
                     jnp.full((16,), 0, jnp.int32),
                     jnp.full((16,), i, jnp.int32)])
        cnt = plsc.load_gather(hist_v, [lk16 >> 7, lk16 & 127])
        inv = 1.0 / jnp.maximum(cnt, 1.0)

        @pl.loop(0, 8)
        def _(r):
          buf[i, pl.ds(r * 16, 16)] = buf[i, pl.ds(r * 16, 16)] * inv

    @pl.loop(0, NBLK // 2)
    def _(t, nb=nb):
      b0 = 2 * t
      b1 = 2 * t + 1

      @pl.when((t > 0) & (b0 < nb))
      def _():
        pltpu.make_async_copy(feat_a, out_hbm.at[gidxb_v.at[0, 0]], osem_a).wait()

      @pl.when(b0 < nb)
      def _():
        pltpu.sync_copy(table_sh.at[idxb_v.at[b0, 0]], feat_a)
        divide(feat_a, b0)
        pltpu.async_copy(feat_a, out_hbm.at[gidxb_v.at[b0, 0]], osem_a)

      @pl.when((t > 0) & (b1 < nb))
      def _():
        pltpu.make_async_copy(feat_b, out_hbm.at[gidxb_v.at[0, 0]], osem_b).wait()

      @pl.when(b1 < nb)
      def _():
        pltpu.sync_copy(table_sh.at[idxb_v.at[b1, 0]], feat_b)
        divide(feat_b, b1)
        pltpu.async_copy(feat_b, out_hbm.at[gidxb_v.at[b1, 0]], osem_b)

    @pl.when(nb >= 1)
    def _():
      pltpu.make_async_copy(feat_a, out_hbm.at[gidxb_v.at[0, 0]], osem_a).wait()

    @pl.when(nb >= 2)
    def _():
      pltpu.make_async_copy(feat_b, out_hbm.at[gidxb_v.at[0, 0]], osem_b).wait()

    plsc.subcore_barrier()


@jax.jit
def _dynmem(styles, addrs, feats):
  mesh = plsc.VectorSubcoreMesh(
      core_axis_name="c", subcore_axis_name="s", num_cores=2, num_subcores=16)
  f32, i32 = jnp.float32, jnp.int32
  call = pl.kernel(
      _body,
      out_type=jax.ShapeDtypeStruct((NR + 1, D), f32),
      mesh=mesh,
      compiler_params=pltpu.CompilerParams(needs_layout_passes=False),
      scratch_types=[
          pltpu.VMEM_SHARED((TROWS, D), f32),
          pltpu.VMEM_SHARED((HROWS, D), f32),
          pltpu.VMEM((CHUNK + 16,), i32),
          pltpu.VMEM((BLK, D), f32),
          pltpu.VMEM((BLK, D), f32),
          pltpu.VMEM((HROWS, D), f32),
          pltpu.VMEM((ARENA,), i32),
          pltpu.VMEM((NBLK, 1, BLK), i32),
          pltpu.VMEM((NBLK, 1, BLK), i32),
          pltpu.VMEM((HROWS,), i32),
          pltpu.SMEM((16,), i32),
          pltpu.SemaphoreType.DMA,
          pltpu.SemaphoreType.DMA,
          pltpu.SemaphoreType.DMA,
          pltpu.SemaphoreType.DMA,
          pltpu.SemaphoreType.DMA,
      ],
  )
  return call(styles, addrs, feats)


def kernel(style_ids, comp_addrs, comp_feats):
  styles = style_ids.astype(jnp.int32)
  addrs = comp_addrs.reshape(-1).astype(jnp.int32)
  feats = jnp.concatenate(
      [comp_feats.reshape(-1, D), jnp.zeros((1, D), jnp.float32)], axis=0)
  out = _dynmem(styles, addrs, feats)
  return out[:NR].reshape(NIN, 3, D)


# speedup vs baseline: 18.1847x; 1.0079x over previous
import jax
import jax.numpy as jnp
from jax import lax
from jax.experimental import pallas as pl
from jax.experimental.pallas import tpu as pltpu
from jax.experimental.pallas import tpu_sc as plsc

N_STY = 256
N_ADR = 371
NKEY = N_STY * N_ADR
NIN = 16384
NR = NIN * 3
D = 128
K = 9600
TROWS = K + 1
NPASS = 5
CHUNK = NR // 16
BLK = 96
NBLK = 32
TRASH_OUT = NR
HROWS = 80
ARENA = CHUNK + 5 * BLK
PAD_ID = CHUNK


def _body(sty_hbm, adr_hbm, feat_hbm, out_hbm,
          table_sh, slab_sh,
          keys_v, feat_a, feat_b, hist_v, arena_v, idxb_v, gidxb_v,
          rowidx_v, secoff_s,
          zsem, asem_a, asem_b, osem_a, osem_b, gsem_a, gsem_b):
  c = lax.axis_index("c")
  s = lax.axis_index("s")
  iota = lax.iota(jnp.int32, 16)
  zf = jnp.zeros((16,), jnp.float32)

  pltpu.sync_copy(sty_hbm.at[pl.ds(s * (CHUNK // 3), CHUNK // 3)],
                  keys_v.at[pl.ds(CHUNK - 1008, CHUNK // 3)])
  pltpu.sync_copy(adr_hbm.at[pl.ds(s * CHUNK, CHUNK)],
                  arena_v.at[pl.ds(0, CHUNK)])

  @pl.loop(0, CHUNK // 16)
  def _(j):
    base = j * 16
    sidx = (CHUNK - 1008) + (base + iota) // 3
    sty16 = plsc.load_gather(keys_v, [sidx])
    a16 = arena_v[pl.ds(base, 16)]
    keys_v[pl.ds(base, 16)] = sty16 * N_ADR + a16

  keys_v[pl.ds(CHUNK, 16)] = jnp.full((16,), -1, jnp.int32)

  @pl.loop(0, ARENA // 16)
  def _(i):
    arena_v[pl.ds(i * 16, 16)] = jnp.full((16,), PAD_ID, jnp.int32)

  @pl.loop(0, HROWS // 16)
  def _(m):
    rowidx_v[pl.ds(m * 16, 16)] = iota + m * 16

  start = jnp.int32(0)
  for pp in range(NPASS):
    rtarget = 2 * pp + c

    def scan_body(j, pos, rtarget=rtarget):
      k16 = keys_v[pl.ds(j * 16, 16)]
      m = (k16 // K) == rtarget
      plsc.store_compressed(arena_v.at[pl.ds(pos, 16)], j * 16 + iota, mask=m)
      return pos + jnp.max(plsc.all_reduce_population_count(m))

    end_real = lax.fori_loop(0, CHUNK // 16, scan_body, start)
    secoff_s[2 * pp] = start
    secoff_s[2 * pp + 1] = end_real
    start = ((end_real + BLK - 1) // BLK) * BLK

  for p in range(NPASS):
    base_key = (2 * p + c) * K
    sec0 = secoff_s[2 * p]
    sec1 = secoff_s[2 * p + 1]
    nb = (sec1 - sec0 + BLK - 1) // BLK

    @pl.loop(0, BLK)
    def _(i):
      @pl.loop(0, 8)
      def _(r):
        feat_a[i, pl.ds(r * 16, 16)] = zf

    @pl.loop(0, NBLK)
    def _(b, base_key=base_key, sec0=sec0, nb=nb):
      @pl.when(b < nb)
      def _():
        @pl.loop(0, BLK // 16)
        def _(j):
          id16 = arena_v[pl.ds(sec0 + b * BLK + j * 16, 16)]
          k16 = plsc.load_gather(keys_v, [id16])
          lk = k16 - base_key
          valid = (lk >= 0) & (lk < K)
          idxb_v[b, 0, pl.ds(j * 16, 16)] = jnp.where(valid, lk, K)
          gidxb_v[b, 0, pl.ds(j * 16, 16)] = jnp.where(
              valid, s * CHUNK + id16, TRASH_OUT)
        pltpu.async_copy(feat_a, table_sh.at[idxb_v.at[b, 0]], zsem)

    pltpu.sync_copy(feat_a.at[pl.ds(0, HROWS // 16)],
                    slab_sh.at[pl.ds(s * (HROWS // 16), HROWS // 16)])

    @pl.loop(0, HROWS)
    def _(i):
      @pl.loop(0, 8)
      def _(r):
        hist_v[i, pl.ds(r * 16, 16)] = zf

    @pl.loop(0, NBLK)
    def _(b, nb=nb):
      @pl.when(b < nb)
      def _():
        pltpu.make_async_copy(feat_a, table_sh.at[idxb_v.at[0, 0]], zsem).wait()

    plsc.subcore_barrier()

    @pl.loop(0, NBLK // 2)
    def _(t, nb=nb):
      b0 = 2 * t
      b1 = 2 * t + 1

      @pl.when((t > 0) & (b0 < nb))
      def _():
        pltpu.make_async_copy(feat_a, table_sh.at[idxb_v.at[0, 0]], asem_a).wait()

      @pl.when(b0 < nb)
      def _():
        pltpu.async_copy(feat_hbm.at[gidxb_v.at[b0, 0]], feat_a, gsem_a)

      @pl.when((t > 0) & (b1 < nb))
      def _():
        pltpu.make_async_copy(feat_b, table_sh.at[idxb_v.at[0, 0]], asem_b).wait()

      @pl.when(b1 < nb)
      def _():
        pltpu.async_copy(feat_hbm.at[gidxb_v.at[b1, 0]], feat_b, gsem_b)

      @pl.when(b0 < nb)
      def _():
        pltpu.make_async_copy(feat_hbm.at[gidxb_v.at[0, 0]], feat_a, gsem_a).wait()
        pltpu.async_copy(feat_a, table_sh.at[idxb_v.at[b0, 0]], asem_a, add=True)

      @pl.when(b1 < nb)
      def _():
        pltpu.make_async_copy(feat_hbm.at[gidxb_v.at[0, 0]], feat_b, gsem_b).wait()
        pltpu.async_copy(feat_b, table_sh.at[idxb_v.at[b1, 0]], asem_b, add=True)

    def hist_body(i, carry, base_key=base_key):
      rid16 = plsc.load_gather(arena_v, [jnp.full((16,), i, jnp.int32)])
      k16 = plsc.load_gather(keys_v, [rid16])
      lk = jnp.max(k16) - base_key
      hi = lk >> 7
      off = lk & 112
      lane = lk & 15
      oh = jnp.where(iota == lane, 1.0, 0.0).astype(jnp.float32)
      hist_v[hi, pl.ds(off, 16)] = hist_v[hi, pl.ds(off, 16)] + oh
      return carry

    lax.fori_loop(sec0, sec1, hist_body, jnp.int32(0))

    @pl.when(nb >= 1)
    def _():
      pltpu.make_async_copy(feat_a, table_sh.at[idxb_v.at[0, 0]], asem_a).wait()

    @pl.when(nb >= 2)
    def _():
      pltpu.make_async_copy(feat_b, table_sh.at[idxb_v.at[0, 0]], asem_b).wait()

    pltpu.sync_copy(hist_v, slab_sh.at[rowidx_v], add=True)
    plsc.subcore_barrier()

    pltpu.sync_copy(slab_sh, hist_v)

    def divide(buf, b):
      @pl.loop(0, BLK)
      def _(i):
        lk16 = plsc.load_gather(
            idxb_v, [jnp.full((16,), b, jnp.int32),
                     jnp.full((16,), 0, jnp.int32),
                     jnp.full((16,), i, jnp.int32)])
        cnt = plsc.load_gather(hist_v, [lk16 >> 7, lk16 & 127])
        inv = 1.0 / jnp.maximum(cnt, 1.0)

        @pl.loop(0, 8)
        def _(r):
          buf[i, pl.ds(r * 16, 16)] = buf[i, pl.ds(r * 16, 16)] * inv

    @pl.loop(0, NBLK // 2)
    def _(t, nb=nb):
      b0 = 2 * t
      b1 = 2 * t + 1

      @pl.when((t > 0) & (b0 < nb))
      def _():
        pltpu.make_async_copy(feat_a, out_hbm.at[gidxb_v.at[0, 0]], osem_a).wait()

      @pl.when(b0 < nb)
      def _():
        pltpu.async_copy(table_sh.at[idxb_v.at[b0, 0]], feat_a, gsem_a)

      @pl.when((t > 0) & (b1 < nb))
      def _():
        pltpu.make_async_copy(feat_b, out_hbm.at[gidxb_v.at[0, 0]], osem_b).wait()

      @pl.when(b1 < nb)
      def _():
        pltpu.async_copy(table_sh.at[idxb_v.at[b1, 0]], feat_b, gsem_b)

      @pl.when(b0 < nb)
      def _():
        pltpu.make_async_copy(table_sh.at[idxb_v.at[0, 0]], feat_a, gsem_a).wait()
        divide(feat_a, b0)
        pltpu.async_copy(feat_a, out_hbm.at[gidxb_v.at[b0, 0]], osem_a)

      @pl.when(b1 < nb)
      def _():
        pltpu.make_async_copy(table_sh.at[idxb_v.at[0, 0]], feat_b, gsem_b).wait()
        divide(feat_b, b1)
        pltpu.async_copy(feat_b, out_hbm.at[gidxb_v.at[b1, 0]], osem_b)

    @pl.when(nb >= 1)
    def _():
      pltpu.make_async_copy(feat_a, out_hbm.at[gidxb_v.at[0, 0]], osem_a).wait()

    @pl.when(nb >= 2)
    def _():
      pltpu.make_async_copy(feat_b, out_hbm.at[gidxb_v.at[0, 0]], osem_b).wait()

    plsc.subcore_barrier()


@jax.jit
def _dynmem(styles, addrs, feats):
  mesh = plsc.VectorSubcoreMesh(
      core_axis_name="c", subcore_axis_name="s", num_cores=2, num_subcores=16)
  f32, i32 = jnp.float32, jnp.int32
  call = pl.kernel(
      _body,
      out_type=jax.ShapeDtypeStruct((NR + 1, D), f32),
      mesh=mesh,
      compiler_params=pltpu.CompilerParams(needs_layout_passes=False),
      scratch_types=[
          pltpu.VMEM_SHARED((TROWS, D), f32),
          pltpu.VMEM_SHARED((HROWS, D), f32),
          pltpu.VMEM((CHUNK + 16,), i32),
          pltpu.VMEM((BLK, D), f32),
          pltpu.VMEM((BLK, D), f32),
          pltpu.VMEM((HROWS, D), f32),
          pltpu.VMEM((ARENA,), i32),
          pltpu.VMEM((NBLK, 1, BLK), i32),
          pltpu.VMEM((NBLK, 1, BLK), i32),
          pltpu.VMEM((HROWS,), i32),
          pltpu.SMEM((16,), i32),
          pltpu.SemaphoreType.DMA,
          pltpu.SemaphoreType.DMA,
          pltpu.SemaphoreType.DMA,
          pltpu.SemaphoreType.DMA,
          pltpu.SemaphoreType.DMA,
          pltpu.SemaphoreType.DMA,
          pltpu.SemaphoreType.DMA,
      ],
  )
  return call(styles, addrs, feats)


def kernel(style_ids, comp_addrs, comp_feats):
  styles = style_ids.astype(jnp.int32)
  addrs = comp_addrs.reshape(-1).astype(jnp.int32)
  feats = jnp.concatenate(
      [comp_feats.reshape(-1, D), jnp.zeros((1, D), jnp.float32)], axis=0)
  out = _dynmem(styles, addrs, feats)
  return out[:NR].reshape(NIN, 3, D)


# speedup vs baseline: 18.1988x; 1.0008x over previous
import jax
import jax.numpy as jnp
from jax import lax
from jax.experimental import pallas as pl
from jax.experimental.pallas import tpu as pltpu
from jax.experimental.pallas import tpu_sc as plsc

N_STY = 256
N_ADR = 371
NKEY = N_STY * N_ADR
NIN = 16384
NR = NIN * 3
D = 128
K = 9600
TROWS = K + 1
NPASS = 5
CHUNK = NR // 16
BLK = 96
NBLK = 32
TRASH_OUT = NR
HROWS = 80
ARENA = CHUNK + 5 * BLK
PAD_ID = CHUNK


def _body(sty_hbm, adr_hbm, feat_hbm, out_hbm,
          table_sh, slab_sh,
          keys_v, feat_a, feat_b, hist_v, arena_v, idxb_v, gidxb_v,
          rowidx_v, secoff_s,
          zsem, asem_a, asem_b, osem_a, osem_b, gsem_a, gsem_b):
  c = lax.axis_index("c")
  s = lax.axis_index("s")
  iota = lax.iota(jnp.int32, 16)
  zf = jnp.zeros((16,), jnp.float32)

  pltpu.sync_copy(sty_hbm.at[pl.ds(s * (CHUNK // 3), CHUNK // 3)],
                  keys_v.at[pl.ds(CHUNK - 1008, CHUNK // 3)])
  pltpu.sync_copy(adr_hbm.at[pl.ds(s * CHUNK, CHUNK)],
                  arena_v.at[pl.ds(0, CHUNK)])

  @pl.loop(0, CHUNK // 16)
  def _(j):
    base = j * 16
    sidx = (CHUNK - 1008) + (base + iota) // 3
    sty16 = plsc.load_gather(keys_v, [sidx])
    a16 = arena_v[pl.ds(base, 16)]
    keys_v[pl.ds(base, 16)] = sty16 * N_ADR + a16

  keys_v[pl.ds(CHUNK, 16)] = jnp.full((16,), -1, jnp.int32)

  @pl.loop(0, ARENA // 16)
  def _(i):
    arena_v[pl.ds(i * 16, 16)] = jnp.full((16,), PAD_ID, jnp.int32)

  @pl.loop(0, HROWS // 16)
  def _(m):
    rowidx_v[pl.ds(m * 16, 16)] = iota + m * 16

  start = jnp.int32(0)
  for pp in range(NPASS):
    rtarget = 2 * pp + c

    def scan_body(j, pos, rtarget=rtarget):
      k16 = keys_v[pl.ds(j * 16, 16)]
      m = (k16 // K) == rtarget
      plsc.store_compressed(arena_v.at[pl.ds(pos, 16)], j * 16 + iota, mask=m)
      return pos + plsc.all_reduce_population_count(m)[0]

    end_real = lax.fori_loop(0, CHUNK // 16, scan_body, start)
    secoff_s[2 * pp] = start
    secoff_s[2 * pp + 1] = end_real
    start = ((end_real + BLK - 1) // BLK) * BLK

  for p in range(NPASS):
    base_key = (2 * p + c) * K
    sec0 = secoff_s[2 * p]
    sec1 = secoff_s[2 * p + 1]
    nb = (sec1 - sec0 + BLK - 1) // BLK

    @pl.loop(0, BLK)
    def _(i):
      @pl.loop(0, 8)
      def _(r):
        feat_a[i, pl.ds(r * 16, 16)] = zf

    @pl.loop(0, NBLK)
    def _(b, base_key=base_key, sec0=sec0, nb=nb):
      @pl.when(b < nb)
      def _():
        @pl.loop(0, BLK // 16)
        def _(j):
          id16 = arena_v[pl.ds(sec0 + b * BLK + j * 16, 16)]
          k16 = plsc.load_gather(keys_v, [id16])
          lk = k16 - base_key
          valid = (lk >= 0) & (lk < K)
          idxb_v[b, 0, pl.ds(j * 16, 16)] = jnp.where(valid, lk, K)
          gidxb_v[b, 0, pl.ds(j * 16, 16)] = jnp.where(
              valid, s * CHUNK + id16, TRASH_OUT)
        pltpu.async_copy(feat_a, table_sh.at[idxb_v.at[b, 0]], zsem)

    pltpu.sync_copy(feat_a.at[pl.ds(0, HROWS // 16)],
                    slab_sh.at[pl.ds(s * (HROWS // 16), HROWS // 16)])

    @pl.loop(0, HROWS)
    def _(i):
      @pl.loop(0, 8)
      def _(r):
        hist_v[i, pl.ds(r * 16, 16)] = zf

    @pl.loop(0, NBLK)
    def _(b, nb=nb):
      @pl.when(b < nb)
      def _():
        pltpu.make_async_copy(feat_a, table_sh.at[idxb_v.at[0, 0]], zsem).wait()

    plsc.subcore_barrier()

    @pl.loop(0, NBLK // 2)
    def _(t, nb=nb):
      b0 = 2 * t
      b1 = 2 * t + 1

      @pl.when((t > 0) & (b0 < nb))
      def _():
        pltpu.make_async_copy(feat_a, table_sh.at[idxb_v.at[0, 0]], asem_a).wait()

      @pl.when(b0 < nb)
      def _():
        pltpu.async_copy(feat_hbm.at[gidxb_v.at[b0, 0]], feat_a, gsem_a)

      @pl.when((t > 0) & (b1 < nb))
      def _():
        pltpu.make_async_copy(feat_b, table_sh.at[idxb_v.at[0, 0]], asem_b).wait()

      @pl.when(b1 < nb)
      def _():
        pltpu.async_copy(feat_hbm.at[gidxb_v.at[b1, 0]], feat_b, gsem_b)

      @pl.when(b0 < nb)
      def _():
        pltpu.make_async_copy(feat_hbm.at[gidxb_v.at[0, 0]], feat_a, gsem_a).wait()
        pltpu.async_copy(feat_a, table_sh.at[idxb_v.at[b0, 0]], asem_a, add=True)

      @pl.when(b1 < nb)
      def _():
        pltpu.make_async_copy(feat_hbm.at[gidxb_v.at[0, 0]], feat_b, gsem_b).wait()
        pltpu.async_copy(feat_b, table_sh.at[idxb_v.at[b1, 0]], asem_b, add=True)

    def hist_body(i, carry, base_key=base_key):
      rid16 = plsc.load_gather(arena_v, [jnp.full((16,), i, jnp.int32)])
      k16 = plsc.load_gather(keys_v, [rid16])
      lk = k16[0] - base_key
      hi = lk >> 7
      off = lk & 112
      lane = lk & 15
      oh = jnp.where(iota == lane, 1.0, 0.0).astype(jnp.float32)
      hist_v[hi, pl.ds(off, 16)] = hist_v[hi, pl.ds(off, 16)] + oh
      return carry

    lax.fori_loop(sec0, sec1, hist_body, jnp.int32(0))

    @pl.when(nb >= 1)
    def _():
      pltpu.make_async_copy(feat_a, table_sh.at[idxb_v.at[0, 0]], asem_a).wait()

    @pl.when(nb >= 2)
    def _():
      pltpu.make_async_copy(feat_b, table_sh.at[idxb_v.at[0, 0]], asem_b).wait()

    pltpu.sync_copy(hist_v, slab_sh.at[rowidx_v], add=True)
    plsc.subcore_barrier()

    pltpu.sync_copy(slab_sh, hist_v)

    def divide(buf, b):
      @pl.loop(0, BLK)
      def _(i):
        lk16 = plsc.load_gather(
            idxb_v, [jnp.full((16,), b, jnp.int32),
                     jnp.full((16,), 0, jnp.int32),
                     jnp.full((16,), i, jnp.int32)])
        cnt = plsc.load_gather(hist_v, [lk16 >> 7, lk16 & 127])
        inv = 1.0 / jnp.maximum(cnt, 1.0)

        @pl.loop(0, 8)
        def _(r):
          buf[i, pl.ds(r * 16, 16)] = buf[i, pl.ds(r * 16, 16)] * inv

    @pl.loop(0, NBLK // 2)
    def _(t, nb=nb):
      b0 = 2 * t
      b1 = 2 * t + 1

      @pl.when((t > 0) & (b0 < nb))
      def _():
        pltpu.make_async_copy(feat_a, out_hbm.at[gidxb_v.at[0, 0]], osem_a).wait()

      @pl.when(b0 < nb)
      def _():
        pltpu.async_copy(table_sh.at[idxb_v.at[b0, 0]], feat_a, gsem_a)

      @pl.when((t > 0) & (b1 < nb))
      def _():
        pltpu.make_async_copy(feat_b, out_hbm.at[gidxb_v.at[0, 0]], osem_b).wait()

      @pl.when(b1 < nb)
      def _():
        pltpu.async_copy(table_sh.at[idxb_v.at[b1, 0]], feat_b, gsem_b)

      @pl.when(b0 < nb)
      def _():
        pltpu.make_async_copy(table_sh.at[idxb_v.at[0, 0]], feat_a, gsem_a).wait()
        divide(feat_a, b0)
        pltpu.async_copy(feat_a, out_hbm.at[gidxb_v.at[b0, 0]], osem_a)

      @pl.when(b1 < nb)
      def _():
        pltpu.make_async_copy(table_sh.at[idxb_v.at[0, 0]], feat_b, gsem_b).wait()
        divide(feat_b, b1)
        pltpu.async_copy(feat_b, out_hbm.at[gidxb_v.at[b1, 0]], osem_b)

    @pl.when(nb >= 1)
    def _():
      pltpu.make_async_copy(feat_a, out_hbm.at[gidxb_v.at[0, 0]], osem_a).wait()

    @pl.when(nb >= 2)
    def _():
      pltpu.make_async_copy(feat_b, out_hbm.at[gidxb_v.at[0, 0]], osem_b).wait()

    plsc.subcore_barrier()


@jax.jit
def _dynmem(styles, addrs, feats):
  mesh = plsc.VectorSubcoreMesh(
      core_axis_name="c", subcore_axis_name="s", num_cores=2, num_subcores=16)
  f32, i32 = jnp.float32, jnp.int32
  call = pl.kernel(
      _body,
      out_type=jax.ShapeDtypeStruct((NR + 1, D), f32),
      mesh=mesh,
      compiler_params=pltpu.CompilerParams(needs_layout_passes=False),
      scratch_types=[
          pltpu.VMEM_SHARED((TROWS, D), f32),
          pltpu.VMEM_SHARED((HROWS, D), f32),
          pltpu.VMEM((CHUNK + 16,), i32),
          pltpu.VMEM((BLK, D), f32),
          pltpu.VMEM((BLK, D), f32),
          pltpu.VMEM((HROWS, D), f32),
          pltpu.VMEM((ARENA,), i32),
          pltpu.VMEM((NBLK, 1, BLK), i32),
          pltpu.VMEM((NBLK, 1, BLK), i32),
          pltpu.VMEM((HROWS,), i32),
          pltpu.SMEM((16,), i32),
          pltpu.SemaphoreType.DMA,
          pltpu.SemaphoreType.DMA,
          pltpu.SemaphoreType.DMA,
          pltpu.SemaphoreType.DMA,
          pltpu.SemaphoreType.DMA,
          pltpu.SemaphoreType.DMA,
          pltpu.SemaphoreType.DMA,
      ],
  )
  return call(styles, addrs, feats)


def kernel(style_ids, comp_addrs, comp_feats):
  styles = style_ids.astype(jnp.int32)
  addrs = comp_addrs.reshape(-1).astype(jnp.int32)
  feats = jnp.concatenate(
      [comp_feats.reshape(-1, D), jnp.zeros((1, D), jnp.float32)], axis=0)
  out = _dynmem(styles, addrs, feats)
  return out[:NR].reshape(NIN, 3, D)
